# baseline (speedup 1.0000x reference)
import functools

import jax
import jax.numpy as jnp
from jax import lax
from jax.experimental import pallas as pl
from jax.experimental.pallas import tpu as pltpu
from jax.experimental.pallas import tpu_sc as plsc

_NC = 2
_NS = 16
_NW = _NC * _NS


def _sc_gather(row_tab, col_tab, fidx_t, cidx_t, rmask_t, cmask_t, B, D):
    S = 2 * B
    CH = S // _NW
    NH = CH // 128

    mesh = plsc.VectorSubcoreMesh(core_axis_name="c", subcore_axis_name="s")

    @functools.partial(
        pl.kernel,
        mesh=mesh,
        out_type=jax.ShapeDtypeStruct((S, D), jnp.float32),
        scratch_types=[
            pltpu.VMEM((CH,), jnp.int32),
            pltpu.VMEM((CH,), jnp.int32),
            pltpu.VMEM((CH,), jnp.float32),
            pltpu.VMEM((CH,), jnp.float32),
            pltpu.VMEM((CH, D), jnp.float32),
            pltpu.VMEM((CH, D), jnp.float32),
            pltpu.VMEM((CH, D), jnp.float32),
            pltpu.SemaphoreType.DMA,
            pltpu.SemaphoreType.DMA,
            pltpu.SemaphoreType.DMA,
            pltpu.SemaphoreType.DMA,
            pltpu.SemaphoreType.DMA,
        ],
    )
    def sc_body(row_hbm, col_hbm, fidx_hbm, cidx_hbm, rmask_hbm, cmask_hbm,
                emb_out,
                fidx_v, cidx_v, rmask_v, cmask_v,
                rows_v, cols_v, emb_v, semr0, semc0, semr1, semc1, semo):
        wid = lax.axis_index("s") * _NC + lax.axis_index("c")
        base = wid * CH

        pltpu.sync_copy(fidx_hbm.at[pl.ds(base, CH)], fidx_v)
        pltpu.sync_copy(cidx_hbm.at[pl.ds(base, CH)], cidx_v)

        sems = [(semr0, semc0), (semr1, semc1)]
        gcps = []
        for h in range(NH):
            hs = pl.ds(h * 128, 128)
            sr, sc = sems[h]
            gcps.append((pltpu.async_copy(row_hbm.at[fidx_v.at[hs]],
                                          rows_v.at[hs], sr),
                         pltpu.async_copy(col_hbm.at[cidx_v.at[hs]],
                                          cols_v.at[hs], sc)))

        pltpu.sync_copy(rmask_hbm.at[pl.ds(base, CH)], rmask_v)
        pltpu.sync_copy(cmask_hbm.at[pl.ds(base, CH)], cmask_v)

        dnums = lax.GatherDimensionNumbers(
            offset_dims=(), collapsed_slice_dims=(0,), start_index_map=(0,))

        def mask_sum_half(h):
            @plsc.parallel_loop(h * 128, (h + 1) * 128, unroll=4)
            def row_body(r):
                g16 = jnp.bitwise_and(r, ~jnp.int32(15))
                rl = jnp.bitwise_and(r, 15)
                gs = pl.ds(pl.multiple_of(g16, 16), 16)
                splat = jnp.broadcast_to(rl, (16,)).reshape(16, 1)
                mr = lax.gather(
                    rmask_v[gs], splat, dnums, (1,),
                    mode=lax.GatherScatterMode.PROMISE_IN_BOUNDS)
                mc = lax.gather(
                    cmask_v[gs], splat, dnums, (1,),
                    mode=lax.GatherScatterMode.PROMISE_IN_BOUNDS)
                for j in range(D // 16):
                    cs = pl.ds(j * 16, 16)
                    emb_v[r, cs] = rows_v[r, cs] * mr + cols_v[r, cs] * mc

        ocps = []
        for h in range(NH):
            cr, cc = gcps[h]
            cr.wait()
            cc.wait()
            mask_sum_half(h)
            hs = pl.ds(h * 128, 128)
            ocps.append(pltpu.async_copy(
                emb_v.at[hs], emb_out.at[pl.ds(base + h * 128, 128)], semo))
        for cp in ocps:
            cp.wait()

    return sc_body(row_tab, col_tab, fidx_t, cidx_t, rmask_t, cmask_t)


def _tc_combine(emb, W, B, D):
    BM = 512
    emb2 = emb.reshape(2, B, D)

    def tc_body(r_ref, w_ref, out_ref):
        w = w_ref[...]
        acc = lax.dot_general(r_ref[0], w[:, :D], (((1,), (1,)), ((), ())),
                              preferred_element_type=jnp.float32)
        acc = acc + lax.dot_general(r_ref[1], w[:, D:], (((1,), (1,)), ((), ())),
                                    preferred_element_type=jnp.float32)
        out_ref[...] = acc

    return pl.pallas_call(
        tc_body,
        grid=(B // BM,),
        in_specs=[
            pl.BlockSpec((2, BM, D), lambda i: (0, i, 0)),
            pl.BlockSpec((D, 2 * D), lambda i: (0, 0)),
        ],
        out_specs=pl.BlockSpec((BM, D), lambda i: (i, 0)),
        out_shape=jax.ShapeDtypeStruct((B, D), jnp.float32),
        compiler_params=pltpu.CompilerParams(
            dimension_semantics=("arbitrary",),
            vmem_limit_bytes=2 * 1024 * 1024,
        ),
    )(emb2, W)


@jax.jit
def _run(encoded_row, encoded_col, W, robot_lot_idx, robot_lot_step, flow,
         num_lot_type, num_step):
    B, R, D = encoded_row.shape
    C = encoded_col.shape[1]

    row_tab = encoded_row.reshape(B * R, D)
    col_tab = encoded_col.reshape(B * C, D)
    lot = robot_lot_idx.astype(jnp.int32)
    step = robot_lot_step.astype(jnp.int32)

    bcol = jnp.arange(B, dtype=jnp.int32)[:, None]
    valid = lot <= num_lot_type
    lf = jnp.where(valid, lot, 0)
    rmask_t = valid.astype(jnp.float32).T.reshape(-1)
    fidx_t = (bcol * R + lf).T.reshape(-1)
    ns = step + 1
    dns = jnp.where(ns > num_step, 0, ns)
    stage = flow[bcol, lf, dns].astype(jnp.int32)
    live = jnp.logical_and(dns > 0,
                           jnp.logical_and(stage >= 1, stage <= C))
    cidx_t = (bcol * C + jnp.where(live, stage - 1, 0)).T.reshape(-1)
    cmask_t = live.astype(jnp.float32).T.reshape(-1)

    emb = _sc_gather(row_tab, col_tab, fidx_t, cidx_t, rmask_t, cmask_t,
                     B, D)
    return _tc_combine(emb, W, B, D)


def kernel(encoded_row, encoded_col, W, robot_lot_idx, robot_lot_step, flow,
           num_lot_type, num_step):
    return _run(encoded_row, encoded_col, W, robot_lot_idx, robot_lot_step,
                flow, num_lot_type, num_step)


# speedup vs baseline: 1.1678x; 1.1678x over previous
import functools

import jax
import jax.numpy as jnp
from jax import lax
from jax.experimental import pallas as pl
from jax.experimental.pallas import tpu as pltpu
from jax.experimental.pallas import tpu_sc as plsc

_NC = 2
_NS = 16
_NW = _NC * _NS


def _sc_gather(row_tab, col_tab, fidx_t, cidx_t, rmask_t, cmask_t, B, D):
    S = 2 * B
    CH = S // _NW
    NH = CH // 128

    mesh = plsc.VectorSubcoreMesh(core_axis_name="c", subcore_axis_name="s")

    @functools.partial(
        pl.kernel,
        mesh=mesh,
        out_type=jax.ShapeDtypeStruct((S, D), jnp.float32),
        scratch_types=[
            pltpu.VMEM((CH,), jnp.int32),
            pltpu.VMEM((CH,), jnp.int32),
            pltpu.VMEM((CH,), jnp.float32),
            pltpu.VMEM((CH,), jnp.float32),
            pltpu.VMEM((CH, D), jnp.float32),
            pltpu.VMEM((CH, D), jnp.float32),
            pltpu.VMEM((CH, D), jnp.float32),
            pltpu.SemaphoreType.DMA,
            pltpu.SemaphoreType.DMA,
            pltpu.SemaphoreType.DMA,
            pltpu.SemaphoreType.DMA,
            pltpu.SemaphoreType.DMA,
        ],
    )
    def sc_body(row_hbm, col_hbm, fidx_hbm, cidx_hbm, rmask_hbm, cmask_hbm,
                emb_out,
                fidx_v, cidx_v, rmask_v, cmask_v,
                rows_v, cols_v, emb_v, semr0, semc0, semr1, semc1, semo):
        wid = lax.axis_index("s") * _NC + lax.axis_index("c")
        base = wid * CH

        pltpu.sync_copy(fidx_hbm.at[pl.ds(base, CH)], fidx_v)
        pltpu.sync_copy(cidx_hbm.at[pl.ds(base, CH)], cidx_v)

        sems = [(semr0, semc0), (semr1, semc1)]
        gcps = []
        for h in range(NH):
            hs = pl.ds(h * 128, 128)
            sr, sc = sems[h]
            gcps.append((pltpu.async_copy(row_hbm.at[fidx_v.at[hs]],
                                          rows_v.at[hs], sr),
                         pltpu.async_copy(col_hbm.at[cidx_v.at[hs]],
                                          cols_v.at[hs], sc)))

        pltpu.sync_copy(rmask_hbm.at[pl.ds(base, CH)], rmask_v)
        pltpu.sync_copy(cmask_hbm.at[pl.ds(base, CH)], cmask_v)

        dnums = lax.GatherDimensionNumbers(
            offset_dims=(), collapsed_slice_dims=(0,), start_index_map=(0,))

        def mask_sum_half(h):
            @plsc.parallel_loop(h * 128, (h + 1) * 128, unroll=4)
            def row_body(r):
                g16 = jnp.bitwise_and(r, ~jnp.int32(15))
                rl = jnp.bitwise_and(r, 15)
                gs = pl.ds(pl.multiple_of(g16, 16), 16)
                splat = jnp.broadcast_to(rl, (16,)).reshape(16, 1)
                mr = lax.gather(
                    rmask_v[gs], splat, dnums, (1,),
                    mode=lax.GatherScatterMode.PROMISE_IN_BOUNDS)
                mc = lax.gather(
                    cmask_v[gs], splat, dnums, (1,),
                    mode=lax.GatherScatterMode.PROMISE_IN_BOUNDS)
                for j in range(D // 16):
                    cs = pl.ds(j * 16, 16)
                    emb_v[r, cs] = rows_v[r, cs] * mr + cols_v[r, cs] * mc

        ocps = []
        for h in range(NH):
            cr, cc = gcps[h]
            cr.wait()
            cc.wait()
            mask_sum_half(h)
            hs = pl.ds(h * 128, 128)
            ocps.append(pltpu.async_copy(
                emb_v.at[hs], emb_out.at[pl.ds(base + h * 128, 128)], semo))
        for cp in ocps:
            cp.wait()

    return sc_body(row_tab, col_tab, fidx_t, cidx_t, rmask_t, cmask_t)


def _tc_combine(emb, W, B, D):
    BM = 512
    NB = B // BM
    emb2 = emb.reshape(2, B, D)

    def tc_body(emb_hbm, w_hbm, out_hbm, eb, wb, ob, wsem, dsem0, dsem1,
                osem0, osem1):
        pltpu.make_async_copy(w_hbm, wb, wsem).start()
        dsems = (dsem0, dsem1)
        osems = (osem0, osem1)

        def in_cp(j):
            s = j % 2
            return pltpu.make_async_copy(
                emb_hbm.at[:, pl.ds(j * BM, BM), :], eb.at[s], dsems[s])

        in_cp(0).start()
        pltpu.make_async_copy(w_hbm, wb, wsem).wait()
        w = wb[...]
        ocps = [None, None]
        for j in range(NB):
            s = j % 2
            if j + 1 < NB:
                in_cp(j + 1).start()
            in_cp(j).wait()
            e0 = eb[s, 0]
            e1 = eb[s, 1]
            acc = lax.dot_general(e0, w[:, :D], (((1,), (1,)), ((), ())),
                                  preferred_element_type=jnp.float32)
            acc = acc + lax.dot_general(e1, w[:, D:],
                                        (((1,), (1,)), ((), ())),
                                        preferred_element_type=jnp.float32)
            if ocps[s] is not None:
                ocps[s].wait()
            ob[s] = acc
            ocps[s] = pltpu.make_async_copy(
                ob.at[s], out_hbm.at[pl.ds(j * BM, BM)], osems[s])
            ocps[s].start()
        for cp in ocps:
            if cp is not None:
                cp.wait()

    return pl.pallas_call(
        tc_body,
        in_specs=[
            pl.BlockSpec(memory_space=pl.ANY),
            pl.BlockSpec(memory_space=pl.ANY),
        ],
        out_specs=pl.BlockSpec(memory_space=pl.ANY),
        out_shape=jax.ShapeDtypeStruct((B, D), jnp.float32),
        scratch_shapes=[
            pltpu.VMEM((2, 2, BM, D), jnp.float32),
            pltpu.VMEM((D, 2 * D), jnp.float32),
            pltpu.VMEM((2, BM, D), jnp.float32),
            pltpu.SemaphoreType.DMA,
            pltpu.SemaphoreType.DMA,
            pltpu.SemaphoreType.DMA,
            pltpu.SemaphoreType.DMA,
            pltpu.SemaphoreType.DMA,
        ],
    )(emb2, W)


@jax.jit
def _run(encoded_row, encoded_col, W, robot_lot_idx, robot_lot_step, flow,
         num_lot_type, num_step):
    B, R, D = encoded_row.shape
    C = encoded_col.shape[1]

    row_tab = encoded_row.reshape(B * R, D)
    col_tab = encoded_col.reshape(B * C, D)
    lot = robot_lot_idx.astype(jnp.int32)
    step = robot_lot_step.astype(jnp.int32)

    lot_t = lot.T.reshape(-1)
    step_t = step.T.reshape(-1)
    b_t = jnp.bitwise_and(jnp.arange(2 * B, dtype=jnp.int32), B - 1)
    valid = lot_t <= num_lot_type
    lf_t = jnp.where(valid, lot_t, 0)
    rmask_t = valid.astype(jnp.float32)
    fidx_t = b_t * R + lf_t
    ns_t = step_t + 1
    dns_t = jnp.where(ns_t > num_step, 0, ns_t)
    stage_t = flow[b_t, lf_t, dns_t].astype(jnp.int32)
    live = jnp.logical_and(dns_t > 0,
                           jnp.logical_and(stage_t >= 1, stage_t <= C))
    cidx_t = b_t * C + jnp.where(live, stage_t - 1, 0)
    cmask_t = live.astype(jnp.float32)

    emb = _sc_gather(row_tab, col_tab, fidx_t, cidx_t, rmask_t, cmask_t,
                     B, D)
    return _tc_combine(emb, W, B, D)


def kernel(encoded_row, encoded_col, W, robot_lot_idx, robot_lot_step, flow,
           num_lot_type, num_step):
    return _run(encoded_row, encoded_col, W, robot_lot_idx, robot_lot_step,
                flow, num_lot_type, num_step)


# speedup vs baseline: 1.1870x; 1.0165x over previous
import functools

import jax
import jax.numpy as jnp
from jax import lax
from jax.experimental import pallas as pl
from jax.experimental.pallas import tpu as pltpu
from jax.experimental.pallas import tpu_sc as plsc

_NC = 2
_NS = 16
_NW = _NC * _NS


def _sc_gather(row_tab, col_tab, fidx_t, cidx_t, rmask_t, cmask_t, B, D):
    S = 2 * B
    CH = S // _NW
    NH = CH // 128

    mesh = plsc.VectorSubcoreMesh(core_axis_name="c", subcore_axis_name="s")

    @functools.partial(
        pl.kernel,
        mesh=mesh,
        out_type=jax.ShapeDtypeStruct((S, D), jnp.float32),
        scratch_types=[
            pltpu.VMEM((CH,), jnp.int32),
            pltpu.VMEM((CH,), jnp.int32),
            pltpu.VMEM((CH,), jnp.float32),
            pltpu.VMEM((CH,), jnp.float32),
            pltpu.VMEM((CH, D), jnp.float32),
            pltpu.VMEM((CH, D), jnp.float32),
            pltpu.VMEM((CH, D), jnp.float32),
            pltpu.SemaphoreType.DMA,
            pltpu.SemaphoreType.DMA,
            pltpu.SemaphoreType.DMA,
            pltpu.SemaphoreType.DMA,
            pltpu.SemaphoreType.DMA,
        ],
    )
    def sc_body(row_hbm, col_hbm, fidx_hbm, cidx_hbm, rmask_hbm, cmask_hbm,
                emb_out,
                fidx_v, cidx_v, rmask_v, cmask_v,
                rows_v, cols_v, emb_v, semr0, semc0, semr1, semc1, semo):
        wid = lax.axis_index("s") * _NC + lax.axis_index("c")
        base = wid * CH

        pltpu.sync_copy(fidx_hbm.at[pl.ds(base, CH)], fidx_v)
        pltpu.sync_copy(cidx_hbm.at[pl.ds(base, CH)], cidx_v)

        sems = [(semr0, semc0), (semr1, semc1)]
        gcps = []
        for h in range(NH):
            hs = pl.ds(h * 128, 128)
            sr, sc = sems[h]
            gcps.append((pltpu.async_copy(row_hbm.at[fidx_v.at[hs]],
                                          rows_v.at[hs], sr),
                         pltpu.async_copy(col_hbm.at[cidx_v.at[hs]],
                                          cols_v.at[hs], sc)))

        pltpu.sync_copy(rmask_hbm.at[pl.ds(base, CH)], rmask_v)
        pltpu.sync_copy(cmask_hbm.at[pl.ds(base, CH)], cmask_v)

        dnums = lax.GatherDimensionNumbers(
            offset_dims=(), collapsed_slice_dims=(0,), start_index_map=(0,))

        def mask_sum_half(h):
            @plsc.parallel_loop(h * 128, (h + 1) * 128, unroll=4)
            def row_body(r):
                g16 = jnp.bitwise_and(r, ~jnp.int32(15))
                rl = jnp.bitwise_and(r, 15)
                gs = pl.ds(pl.multiple_of(g16, 16), 16)
                splat = jnp.broadcast_to(rl, (16,)).reshape(16, 1)
                mr = lax.gather(
                    rmask_v[gs], splat, dnums, (1,),
                    mode=lax.GatherScatterMode.PROMISE_IN_BOUNDS)
                mc = lax.gather(
                    cmask_v[gs], splat, dnums, (1,),
                    mode=lax.GatherScatterMode.PROMISE_IN_BOUNDS)
                for j in range(D // 16):
                    cs = pl.ds(j * 16, 16)
                    emb_v[r, cs] = rows_v[r, cs] * mr + cols_v[r, cs] * mc

        ocps = []
        for h in range(NH):
            cr, cc = gcps[h]
            cr.wait()
            cc.wait()
            mask_sum_half(h)
            hs = pl.ds(h * 128, 128)
            ocps.append(pltpu.async_copy(
                emb_v.at[hs], emb_out.at[pl.ds(base + h * 128, 128)], semo))
        for cp in ocps:
            cp.wait()

    return sc_body(row_tab, col_tab, fidx_t, cidx_t, rmask_t, cmask_t)


def _tc_combine(emb, W, B, D):
    BM = 512
    emb2 = emb.reshape(2, B, D)

    def tc_body(r_ref, w_ref, out_ref):
        w = w_ref[...]
        acc = lax.dot_general(r_ref[0], w[:, :D], (((1,), (1,)), ((), ())),
                              preferred_element_type=jnp.float32)
        acc = acc + lax.dot_general(r_ref[1], w[:, D:], (((1,), (1,)), ((), ())),
                                    preferred_element_type=jnp.float32)
        out_ref[...] = acc

    return pl.pallas_call(
        tc_body,
        grid=(B // BM,),
        in_specs=[
            pl.BlockSpec((2, BM, D), lambda i: (0, i, 0)),
            pl.BlockSpec((D, 2 * D), lambda i: (0, 0)),
        ],
        out_specs=pl.BlockSpec((BM, D), lambda i: (i, 0)),
        out_shape=jax.ShapeDtypeStruct((B, D), jnp.float32),
        compiler_params=pltpu.CompilerParams(
            dimension_semantics=("arbitrary",),
            vmem_limit_bytes=2 * 1024 * 1024,
        ),
    )(emb2, W)


@jax.jit
def _run(encoded_row, encoded_col, W, robot_lot_idx, robot_lot_step, flow,
         num_lot_type, num_step):
    B, R, D = encoded_row.shape
    C = encoded_col.shape[1]

    row_tab = encoded_row.reshape(B * R, D)
    col_tab = encoded_col.reshape(B * C, D)
    lot = robot_lot_idx.astype(jnp.int32)
    step = robot_lot_step.astype(jnp.int32)

    lot_t = lot.T.reshape(-1)
    step_t = step.T.reshape(-1)
    b_t = jnp.bitwise_and(jnp.arange(2 * B, dtype=jnp.int32), B - 1)
    valid = lot_t <= num_lot_type
    lf_t = jnp.where(valid, lot_t, 0)
    rmask_t = valid.astype(jnp.float32)
    fidx_t = b_t * R + lf_t
    ns_t = step_t + 1
    dns_t = jnp.where(ns_t > num_step, 0, ns_t)
    stage_t = flow[b_t, lf_t, dns_t].astype(jnp.int32)
    live = jnp.logical_and(dns_t > 0,
                           jnp.logical_and(stage_t >= 1, stage_t <= C))
    cidx_t = b_t * C + jnp.where(live, stage_t - 1, 0)
    cmask_t = live.astype(jnp.float32)

    emb = _sc_gather(row_tab, col_tab, fidx_t, cidx_t, rmask_t, cmask_t,
                     B, D)
    return _tc_combine(emb, W, B, D)


def kernel(encoded_row, encoded_col, W, robot_lot_idx, robot_lot_step, flow,
           num_lot_type, num_step):
    return _run(encoded_row, encoded_col, W, robot_lot_idx, robot_lot_step,
                flow, num_lot_type, num_step)
